# baseline (speedup 1.0000x reference)
import functools

import jax
import jax.numpy as jnp
from jax import lax
from jax.experimental import pallas as pl
from jax.experimental.pallas import tpu as pltpu
from jax.experimental.pallas import tpu_sc as plsc

QB = 64
LG = 128
ROW = 8
NWORK = 32
NQ = 1024
BIGF = 3.0e38


def _topk_body(xq_ref, xkT_ref, idx_ref, d2_ref):
    xq = xq_ref[...]
    xkT = xkT_ref[...]
    KP = xkT.shape[1]
    NG = KP // LG
    q2 = jnp.sum(xq * xq, axis=1, keepdims=True)
    k2 = jnp.sum(xkT * xkT, axis=0, keepdims=True)
    qk = lax.dot_general(
        xq, xkT, (((1,), (0,)), ((), ())),
        preferred_element_type=jnp.float32,
    )
    d2_ref[...] = q2 + k2 - 2.0 * qk

    def insert(g, st):
        s0, s1, s2, s3, s4, i0, i1, i2, i3, i4 = st
        x = d2_ref[:, pl.ds(pl.multiple_of(g * LG, LG), LG)]
        xi = jnp.full((QB, LG), 1.0, jnp.float32) * g.astype(jnp.float32)
        s, i_ = [s0, s1, s2, s3, s4], [i0, i1, i2, i3, i4]
        for k in range(5):
            swap = x < s[k]
            ns = jnp.minimum(s[k], x)
            nx = jnp.maximum(s[k], x)
            nik = jnp.where(swap, xi, i_[k])
            nxi = jnp.where(swap, i_[k], xi)
            s[k], x, i_[k], xi = ns, nx, nik, nxi
        return tuple(s + i_)

    inf2 = jnp.full((QB, LG), jnp.inf, jnp.float32)
    zero2 = jnp.zeros((QB, LG), jnp.float32)
    st = lax.fori_loop(0, NG, insert,
                       (inf2, inf2, inf2, inf2, inf2,
                        zero2, zero2, zero2, zero2, zero2),
                       unroll=4)
    s, i_ = list(st[:5]), list(st[5:])

    lane_f = lax.broadcasted_iota(jnp.int32, (QB, LG), 1).astype(jnp.float32)
    ci = []
    for t in range(5):
        m = jnp.min(s[0], axis=1, keepdims=True)
        gl = i_[0] * float(LG) + lane_f
        cand = jnp.where(s[0] == m, gl, BIGF)
        gsel = jnp.min(cand, axis=1, keepdims=True)
        ci.append(gsel)
        if t < 4:
            pop = cand == gsel
            for k in range(4):
                s[k] = jnp.where(pop, s[k + 1], s[k])
                i_[k] = jnp.where(pop, i_[k + 1], i_[k])
            s[4] = jnp.where(pop, jnp.inf, s[4])

    idx_ref[...] = jnp.concatenate(
        ci + [jnp.zeros((QB, 3), jnp.float32)], axis=1).astype(jnp.int32)


def _topk_indices(X_test, XT):
    Q, D = X_test.shape
    KP = XT.shape[1]
    return pl.pallas_call(
        _topk_body,
        grid=(Q // QB,),
        in_specs=[
            pl.BlockSpec((QB, D), lambda i: (i, 0)),
            pl.BlockSpec((D, KP), lambda i: (0, 0)),
        ],
        out_specs=pl.BlockSpec((QB, ROW), lambda i: (i, 0)),
        out_shape=jax.ShapeDtypeStruct((Q, ROW), jnp.int32),
        scratch_shapes=[pltpu.VMEM((QB, KP), jnp.float32)],
    )(X_test, XT)


QPW = NQ // NWORK
NSEL = 5


def _sc_body(idx_hbm, y_hbm, out_hbm, idx_v, yv, pv, sem):
    c = lax.axis_index("c")
    s = lax.axis_index("s")
    w = s * 2 + c
    base = w * QPW
    for t in range(NSEL):
        pltpu.sync_copy(idx_hbm.at[pl.ds(t * NQ + base, QPW)],
                        idx_v.at[pl.ds(t * QPW, QPW)])
    for t in range(NSEL):
        pltpu.async_copy(
            y_hbm.at[idx_v.at[pl.ds(t * QPW, QPW)]],
            yv.at[pl.ds(t * QPW, QPW)], sem).wait()
    for h in range(QPW // 16):
        acc = jnp.zeros((16,), jnp.float32)
        for t in range(NSEL):
            acc = acc + yv[pl.ds(t * QPW + h * 16, 16)]
        pv[pl.ds(h * 16, 16)] = acc * 0.2
    pltpu.sync_copy(pv, out_hbm.at[pl.ds(base, QPW)])


@functools.cache
def _sc_gather_mean():
    return pl.kernel(
        _sc_body,
        out_type=jax.ShapeDtypeStruct((NQ,), jnp.float32),
        mesh=plsc.VectorSubcoreMesh(core_axis_name="c", subcore_axis_name="s"),
        scratch_types=[
            pltpu.VMEM((NSEL * QPW,), jnp.int32),
            pltpu.VMEM((NSEL * QPW,), jnp.float32),
            pltpu.VMEM((QPW,), jnp.float32),
            pltpu.SemaphoreType.DMA,
        ],
    )


def kernel(X_test, X_train, y_train):
    Q, D = X_test.shape
    K = X_train.shape[0]
    KP = ((K + LG - 1) // LG) * LG
    pad = KP - K
    Xp = jnp.concatenate(
        [X_train, jnp.full((pad, D), 1e18, jnp.float32)], axis=0)
    idx = _topk_indices(X_test, Xp.T)
    idx_t = idx[:, :NSEL].T.reshape(NSEL * Q)
    preds = _sc_gather_mean()(idx_t, y_train)
    return preds


# speedup vs baseline: 1.1358x; 1.1358x over previous
import functools

import jax
import jax.numpy as jnp
from jax import lax
from jax.experimental import pallas as pl
from jax.experimental.pallas import tpu as pltpu
from jax.experimental.pallas import tpu_sc as plsc

QB = 32
LG = 128
ROW = 8
NWORK = 32
NQ = 1024
BIGF = 3.0e38


def _topk_body(xq_ref, xkT_ref, idx_ref, d2_ref):
    xq = xq_ref[...]
    xkT = xkT_ref[...]
    KP = xkT.shape[1]
    NG = KP // LG
    q2 = jnp.sum(xq * xq, axis=1, keepdims=True)
    k2 = jnp.sum(xkT * xkT, axis=0, keepdims=True)
    qk = lax.dot_general(
        xq, xkT, (((1,), (0,)), ((), ())),
        preferred_element_type=jnp.float32,
    )
    d2_ref[...] = q2 + k2 - 2.0 * qk

    def insert(g, st):
        s0, s1, s2, s3, s4, i0, i1, i2, i3, i4 = st
        x = d2_ref[:, pl.ds(pl.multiple_of(g * LG, LG), LG)]
        xi = g.astype(jnp.float32)
        s, i_ = [s0, s1, s2, s3, s4], [i0, i1, i2, i3, i4]
        for k in range(5):
            swap = x < s[k]
            ns = jnp.minimum(s[k], x)
            nx = jnp.maximum(s[k], x)
            nik = jnp.where(swap, xi, i_[k])
            nxi = jnp.where(swap, i_[k], xi)
            s[k], x, i_[k], xi = ns, nx, nik, nxi
        return tuple(s + i_)

    inf2 = jnp.full((QB, LG), jnp.inf, jnp.float32)
    zero2 = jnp.zeros((QB, LG), jnp.float32)
    st = lax.fori_loop(0, NG, insert,
                       (inf2, inf2, inf2, inf2, inf2,
                        zero2, zero2, zero2, zero2, zero2),
                       unroll=8)
    s, i_ = list(st[:5]), list(st[5:])

    lane_f = lax.broadcasted_iota(jnp.int32, (QB, LG), 1).astype(jnp.float32)
    ci = []
    for t in range(5):
        m = jnp.min(s[0], axis=1, keepdims=True)
        gl = i_[0] * float(LG) + lane_f
        cand = jnp.where(s[0] == m, gl, BIGF)
        gsel = jnp.min(cand, axis=1, keepdims=True)
        ci.append(gsel)
        if t < 4:
            pop = cand == gsel
            for k in range(4):
                s[k] = jnp.where(pop, s[k + 1], s[k])
                i_[k] = jnp.where(pop, i_[k + 1], i_[k])
            s[4] = jnp.where(pop, jnp.inf, s[4])

    idx_ref[...] = jnp.concatenate(
        ci + [jnp.zeros((QB, 3), jnp.float32)], axis=1).astype(jnp.int32)


def _topk_indices(X_test, XT):
    Q, D = X_test.shape
    KP = XT.shape[1]
    return pl.pallas_call(
        _topk_body,
        grid=(Q // QB,),
        in_specs=[
            pl.BlockSpec((QB, D), lambda i: (i, 0)),
            pl.BlockSpec((D, KP), lambda i: (0, 0)),
        ],
        out_specs=pl.BlockSpec((QB, ROW), lambda i: (i, 0)),
        out_shape=jax.ShapeDtypeStruct((Q, ROW), jnp.int32),
        scratch_shapes=[pltpu.VMEM((QB, KP), jnp.float32)],
    )(X_test, XT)


QPW = NQ // NWORK
NSEL = 5


def _sc_body(idx_hbm, y_hbm, out_hbm, idx_v, yv, pv, sem):
    c = lax.axis_index("c")
    s = lax.axis_index("s")
    w = s * 2 + c
    base = w * QPW
    for t in range(NSEL):
        pltpu.sync_copy(idx_hbm.at[pl.ds(t * NQ + base, QPW)],
                        idx_v.at[pl.ds(t * QPW, QPW)])
    for t in range(NSEL):
        pltpu.async_copy(
            y_hbm.at[idx_v.at[pl.ds(t * QPW, QPW)]],
            yv.at[pl.ds(t * QPW, QPW)], sem).wait()
    for h in range(QPW // 16):
        acc = jnp.zeros((16,), jnp.float32)
        for t in range(NSEL):
            acc = acc + yv[pl.ds(t * QPW + h * 16, 16)]
        pv[pl.ds(h * 16, 16)] = acc * 0.2
    pltpu.sync_copy(pv, out_hbm.at[pl.ds(base, QPW)])


@functools.cache
def _sc_gather_mean():
    return pl.kernel(
        _sc_body,
        out_type=jax.ShapeDtypeStruct((NQ,), jnp.float32),
        mesh=plsc.VectorSubcoreMesh(core_axis_name="c", subcore_axis_name="s"),
        scratch_types=[
            pltpu.VMEM((NSEL * QPW,), jnp.int32),
            pltpu.VMEM((NSEL * QPW,), jnp.float32),
            pltpu.VMEM((QPW,), jnp.float32),
            pltpu.SemaphoreType.DMA,
        ],
    )


def kernel(X_test, X_train, y_train):
    Q, D = X_test.shape
    K = X_train.shape[0]
    KP = ((K + LG - 1) // LG) * LG
    pad = KP - K
    Xp = jnp.concatenate(
        [X_train, jnp.full((pad, D), 1e18, jnp.float32)], axis=0)
    idx = _topk_indices(X_test, Xp.T)
    idx_t = idx[:, :NSEL].T.reshape(NSEL * Q)
    preds = _sc_gather_mean()(idx_t, y_train)
    return preds


# speedup vs baseline: 1.1833x; 1.0418x over previous
import functools

import jax
import jax.numpy as jnp
from jax import lax
from jax.experimental import pallas as pl
from jax.experimental.pallas import tpu as pltpu
from jax.experimental.pallas import tpu_sc as plsc

QB = 32
LG = 128
ROW = 8
NWORK = 32
NQ = 1024
BIGF = 3.0e38


def _topk_body(xq_ref, xkT_ref, idx_ref, d2_ref):
    xq = xq_ref[...]
    xkT = xkT_ref[...]
    KP = xkT.shape[1]
    NG = KP // LG
    q2 = jnp.sum(xq * xq, axis=1, keepdims=True)
    k2 = jnp.sum(xkT * xkT, axis=0, keepdims=True)
    qk = lax.dot_general(
        xq, xkT, (((1,), (0,)), ((), ())),
        preferred_element_type=jnp.float32,
    )
    d2_ref[...] = q2 + k2 - 2.0 * qk

    def insert(g, st):
        s0, s1, s2, s3, s4, i0, i1, i2, i3, i4 = st
        x = d2_ref[:, pl.ds(pl.multiple_of(g * LG, LG), LG)]
        xi = g.astype(jnp.float32)
        s, i_ = [s0, s1, s2, s3, s4], [i0, i1, i2, i3, i4]
        for k in range(5):
            swap = x < s[k]
            ns = jnp.minimum(s[k], x)
            nx = jnp.maximum(s[k], x)
            nik = jnp.where(swap, xi, i_[k])
            nxi = jnp.where(swap, i_[k], xi)
            s[k], x, i_[k], xi = ns, nx, nik, nxi
        return tuple(s + i_)

    inf2 = jnp.full((QB, LG), jnp.inf, jnp.float32)
    zero2 = jnp.zeros((QB, LG), jnp.float32)
    st = lax.fori_loop(0, NG, insert,
                       (inf2, inf2, inf2, inf2, inf2,
                        zero2, zero2, zero2, zero2, zero2),
                       unroll=16)
    s, i_ = list(st[:5]), list(st[5:])

    lane_f = lax.broadcasted_iota(jnp.int32, (QB, LG), 1).astype(jnp.float32)
    ci = []
    for t in range(5):
        m = jnp.min(s[0], axis=1, keepdims=True)
        gl = i_[0] * float(LG) + lane_f
        cand = jnp.where(s[0] == m, gl, BIGF)
        gsel = jnp.min(cand, axis=1, keepdims=True)
        ci.append(gsel)
        if t < 4:
            pop = cand == gsel
            for k in range(4):
                s[k] = jnp.where(pop, s[k + 1], s[k])
                i_[k] = jnp.where(pop, i_[k + 1], i_[k])
            s[4] = jnp.where(pop, jnp.inf, s[4])

    idx_ref[...] = jnp.concatenate(
        ci + [jnp.zeros((QB, 3), jnp.float32)], axis=1).astype(jnp.int32)


def _topk_indices(X_test, XT):
    Q, D = X_test.shape
    KP = XT.shape[1]
    return pl.pallas_call(
        _topk_body,
        grid=(Q // QB,),
        in_specs=[
            pl.BlockSpec((QB, D), lambda i: (i, 0)),
            pl.BlockSpec((D, KP), lambda i: (0, 0)),
        ],
        out_specs=pl.BlockSpec((QB, ROW), lambda i: (i, 0)),
        out_shape=jax.ShapeDtypeStruct((Q, ROW), jnp.int32),
        scratch_shapes=[pltpu.VMEM((QB, KP), jnp.float32)],
    )(X_test, XT)


QPW = NQ // NWORK
NSEL = 5


def _sc_body(idx_hbm, y_hbm, out_hbm, idx_v, yv, pv, sem):
    c = lax.axis_index("c")
    s = lax.axis_index("s")
    w = s * 2 + c
    base = w * QPW
    for t in range(NSEL):
        pltpu.sync_copy(idx_hbm.at[pl.ds(t * NQ + base, QPW)],
                        idx_v.at[pl.ds(t * QPW, QPW)])
    for t in range(NSEL):
        pltpu.async_copy(
            y_hbm.at[idx_v.at[pl.ds(t * QPW, QPW)]],
            yv.at[pl.ds(t * QPW, QPW)], sem).wait()
    for h in range(QPW // 16):
        acc = jnp.zeros((16,), jnp.float32)
        for t in range(NSEL):
            acc = acc + yv[pl.ds(t * QPW + h * 16, 16)]
        pv[pl.ds(h * 16, 16)] = acc * 0.2
    pltpu.sync_copy(pv, out_hbm.at[pl.ds(base, QPW)])


@functools.cache
def _sc_gather_mean():
    return pl.kernel(
        _sc_body,
        out_type=jax.ShapeDtypeStruct((NQ,), jnp.float32),
        mesh=plsc.VectorSubcoreMesh(core_axis_name="c", subcore_axis_name="s"),
        scratch_types=[
            pltpu.VMEM((NSEL * QPW,), jnp.int32),
            pltpu.VMEM((NSEL * QPW,), jnp.float32),
            pltpu.VMEM((QPW,), jnp.float32),
            pltpu.SemaphoreType.DMA,
        ],
    )


def kernel(X_test, X_train, y_train):
    Q, D = X_test.shape
    K = X_train.shape[0]
    KP = ((K + LG - 1) // LG) * LG
    pad = KP - K
    Xp = jnp.concatenate(
        [X_train, jnp.full((pad, D), 1e18, jnp.float32)], axis=0)
    idx = _topk_indices(X_test, Xp.T)
    idx_t = idx[:, :NSEL].T.reshape(NSEL * Q)
    preds = _sc_gather_mean()(idx_t, y_train)
    return preds
